# baseline (device time: 46242 ns/iter reference)
import jax
import jax.numpy as jnp
from jax import lax
from jax.experimental import pallas as pl
from jax.experimental.pallas import tpu as pltpu

N_DEV = 8
MASKS = (1, 3, 4)


def kernel(A, B):
    m, k = A.shape
    k2, n = B.shape

    def body(a_ref, b_ref, out_ref, comm_ref, send_sems, recv_sems):
        my_pos = lax.axis_index("i")

        barrier_sem = pltpu.get_barrier_semaphore()
        for mask in MASKS:
            pl.semaphore_signal(
                barrier_sem,
                inc=1,
                device_id=(my_pos ^ mask,),
                device_id_type=pl.DeviceIdType.MESH,
            )
        pl.semaphore_wait(barrier_sem, len(MASKS))

        out_ref[...] = jnp.dot(
            a_ref[...], b_ref[...], preferred_element_type=jnp.float32
        )

        for r, mask in enumerate(MASKS):
            partner = my_pos ^ mask
            rdma = pltpu.make_async_remote_copy(
                src_ref=out_ref,
                dst_ref=comm_ref.at[r],
                send_sem=send_sems.at[r],
                recv_sem=recv_sems.at[r],
                device_id=(partner,),
                device_id_type=pl.DeviceIdType.MESH,
            )
            rdma.start()
            rdma.wait()
            out_ref[...] += comm_ref[r]

        out_ref[...] = jnp.maximum(out_ref[...], 0.0)

    return pl.pallas_call(
        body,
        out_shape=jax.ShapeDtypeStruct((m, n), jnp.float32),
        in_specs=[
            pl.BlockSpec(memory_space=pltpu.VMEM),
            pl.BlockSpec(memory_space=pltpu.VMEM),
        ],
        out_specs=pl.BlockSpec(memory_space=pltpu.VMEM),
        scratch_shapes=[
            pltpu.VMEM((len(MASKS), m, n), jnp.float32),
            pltpu.SemaphoreType.DMA((len(MASKS),)),
            pltpu.SemaphoreType.DMA((len(MASKS),)),
        ],
        compiler_params=pltpu.CompilerParams(collective_id=0),
    )(A, B)


# device time: 36284 ns/iter; 1.2744x vs baseline; 1.2744x over previous
import jax
import jax.numpy as jnp
from jax import lax
from jax.experimental import pallas as pl
from jax.experimental.pallas import tpu as pltpu

N_DEV = 8
MASKS = (1, 3, 4)


def kernel(A, B):
    m, k = A.shape
    k2, n = B.shape
    half, quart, eighth = m // 2, m // 4, m // 8

    def body(a_ref, b_ref, out_ref, comm_ref, send_sems, recv_sems):
        my_pos = lax.axis_index("i")
        b0 = my_pos & 1
        b1 = (my_pos >> 1) & 1
        b2 = (my_pos >> 2) & 1
        f1 = b0 ^ b1
        f2 = b1
        f3 = b2
        partners = [my_pos ^ mask for mask in MASKS]

        barrier_sem = pltpu.get_barrier_semaphore()
        for p in partners:
            pl.semaphore_signal(
                barrier_sem, inc=1,
                device_id=(p,), device_id_type=pl.DeviceIdType.MESH,
            )
        pl.semaphore_wait(barrier_sem, len(MASKS))

        pending = []

        def exch(rnd, src_slice, dst_ref, partner):
            rdma = pltpu.make_async_remote_copy(
                src_ref=out_ref.at[src_slice, :],
                dst_ref=dst_ref,
                send_sem=send_sems.at[rnd],
                recv_sem=recv_sems.at[rnd],
                device_id=(partner,),
                device_id_type=pl.DeviceIdType.MESH,
            )
            rdma.start()
            pending.append(rdma)
            return rdma

        keep1 = f1 * half
        send1 = (1 - f1) * half
        out_ref[pl.ds(send1, half), :] = jnp.dot(
            a_ref[pl.ds(send1, half), :], b_ref[...],
            preferred_element_type=jnp.float32,
        )
        r1 = exch(0, pl.ds(send1, half), comm_ref.at[0, pl.ds(0, half), :],
                  partners[0])
        out_ref[pl.ds(keep1, half), :] = jnp.dot(
            a_ref[pl.ds(keep1, half), :], b_ref[...],
            preferred_element_type=jnp.float32,
        )
        r1.wait_recv()

        keep2 = keep1 + f2 * quart
        send2 = keep1 + (1 - f2) * quart
        out_ref[pl.ds(send2, quart), :] += comm_ref[0, pl.ds((1 - f2) * quart, quart), :]
        r2 = exch(1, pl.ds(send2, quart), comm_ref.at[1, pl.ds(0, quart), :],
                  partners[1])
        out_ref[pl.ds(keep2, quart), :] += comm_ref[0, pl.ds(f2 * quart, quart), :]
        r2.wait_recv()

        keep3 = keep2 + f3 * eighth
        send3 = keep2 + (1 - f3) * eighth
        out_ref[pl.ds(send3, eighth), :] += comm_ref[1, pl.ds((1 - f3) * eighth, eighth), :]
        r3 = exch(2, pl.ds(send3, eighth), comm_ref.at[2, pl.ds(0, eighth), :],
                  partners[2])
        out_ref[pl.ds(keep3, eighth), :] += comm_ref[1, pl.ds(f3 * eighth, eighth), :]
        r3.wait_recv()

        out_ref[pl.ds(keep3, eighth), :] = jnp.maximum(
            out_ref[pl.ds(keep3, eighth), :] + comm_ref[2, pl.ds(0, eighth), :],
            0.0,
        )

        r4 = exch(3, pl.ds(keep3, eighth),
                  out_ref.at[pl.ds(keep3, eighth), :], partners[2])
        r4.wait_recv()
        r5 = exch(4, pl.ds(keep2, quart),
                  out_ref.at[pl.ds(keep2, quart), :], partners[1])
        r5.wait_recv()
        r6 = exch(5, pl.ds(keep1, half),
                  out_ref.at[pl.ds(keep1, half), :], partners[0])
        r6.wait_recv()

        for rdma in pending:
            rdma.wait_send()

    return pl.pallas_call(
        body,
        out_shape=jax.ShapeDtypeStruct((m, n), jnp.float32),
        in_specs=[
            pl.BlockSpec(memory_space=pltpu.VMEM),
            pl.BlockSpec(memory_space=pltpu.VMEM),
        ],
        out_specs=pl.BlockSpec(memory_space=pltpu.VMEM),
        scratch_shapes=[
            pltpu.VMEM((3, m // 2, n), jnp.float32),
            pltpu.SemaphoreType.DMA((6,)),
            pltpu.SemaphoreType.DMA((6,)),
        ],
        compiler_params=pltpu.CompilerParams(collective_id=0),
    )(A, B)


# device time: 23466 ns/iter; 1.9706x vs baseline; 1.5462x over previous
import jax
import jax.numpy as jnp
from jax import lax
from jax.experimental import pallas as pl
from jax.experimental.pallas import tpu as pltpu

N_DEV = 8
PEERS = (6, 2, 5, 7, 1, 3, 4)


def kernel(A, B):
    m, k = A.shape
    k2, n = B.shape
    eighth = m // N_DEV

    def eighth_start(pos):
        b0 = pos & 1
        b1 = (pos >> 1) & 1
        b2 = (pos >> 2) & 1
        return ((b0 ^ b1) * 4 + b1 * 2 + b2) * eighth

    def body(a_ref, b_ref, out_ref, comm_ref,
             rs_send, rs_recv, ag_send, ag_recv):
        my_pos = lax.axis_index("i")
        my_e = eighth_start(my_pos)

        barrier_sem = pltpu.get_barrier_semaphore()
        for d in PEERS:
            pl.semaphore_signal(
                barrier_sem, inc=1,
                device_id=(my_pos ^ d,), device_id_type=pl.DeviceIdType.MESH,
            )
        pl.semaphore_wait(barrier_sem, len(PEERS))

        rs = {}
        for d in PEERS:
            t = my_pos ^ d
            te = eighth_start(t)
            out_ref[pl.ds(te, eighth), :] = jnp.dot(
                a_ref[pl.ds(te, eighth), :], b_ref[...],
                preferred_element_type=jnp.float32,
            )
            rs[d] = pltpu.make_async_remote_copy(
                src_ref=out_ref.at[pl.ds(te, eighth), :],
                dst_ref=comm_ref.at[d - 1],
                send_sem=rs_send.at[d - 1],
                recv_sem=rs_recv.at[d - 1],
                device_id=(t,),
                device_id_type=pl.DeviceIdType.MESH,
            )
            rs[d].start()
        out_ref[pl.ds(my_e, eighth), :] = jnp.dot(
            a_ref[pl.ds(my_e, eighth), :], b_ref[...],
            preferred_element_type=jnp.float32,
        )

        for d in PEERS:
            rs[d].wait_recv()
            out_ref[pl.ds(my_e, eighth), :] += comm_ref[d - 1]
        out_ref[pl.ds(my_e, eighth), :] = jnp.maximum(
            out_ref[pl.ds(my_e, eighth), :], 0.0
        )

        ag = {}
        for d in PEERS:
            t = my_pos ^ d
            ag[d] = pltpu.make_async_remote_copy(
                src_ref=out_ref.at[pl.ds(my_e, eighth), :],
                dst_ref=out_ref.at[pl.ds(my_e, eighth), :],
                send_sem=ag_send.at[d - 1],
                recv_sem=ag_recv.at[d - 1],
                device_id=(t,),
                device_id_type=pl.DeviceIdType.MESH,
            )
            ag[d].start()
        for d in PEERS:
            recv = pltpu.make_async_remote_copy(
                src_ref=out_ref.at[pl.ds(my_e, eighth), :],
                dst_ref=out_ref.at[pl.ds(eighth_start(my_pos ^ d), eighth), :],
                send_sem=ag_send.at[d - 1],
                recv_sem=ag_recv.at[d - 1],
                device_id=(my_pos ^ d,),
                device_id_type=pl.DeviceIdType.MESH,
            )
            recv.wait_recv()

        for d in PEERS:
            rs[d].wait_send()
            ag[d].wait_send()

    return pl.pallas_call(
        body,
        out_shape=jax.ShapeDtypeStruct((m, n), jnp.float32),
        in_specs=[
            pl.BlockSpec(memory_space=pltpu.VMEM),
            pl.BlockSpec(memory_space=pltpu.VMEM),
        ],
        out_specs=pl.BlockSpec(memory_space=pltpu.VMEM),
        scratch_shapes=[
            pltpu.VMEM((N_DEV - 1, eighth, n), jnp.float32),
            pltpu.SemaphoreType.DMA((N_DEV - 1,)),
            pltpu.SemaphoreType.DMA((N_DEV - 1,)),
            pltpu.SemaphoreType.DMA((N_DEV - 1,)),
            pltpu.SemaphoreType.DMA((N_DEV - 1,)),
        ],
        compiler_params=pltpu.CompilerParams(collective_id=0),
    )(A, B)


# device time: 22956 ns/iter; 2.0144x vs baseline; 1.0222x over previous
import jax
import jax.numpy as jnp
from jax import lax
from jax.experimental import pallas as pl
from jax.experimental.pallas import tpu as pltpu

N_DEV = 8
PEERS = (6, 2, 5, 7, 1, 3, 4)
N_CHUNK = 4


def kernel(A, B):
    m, k = A.shape
    k2, n = B.shape
    eighth = m // N_DEV
    chunk = eighth // N_CHUNK

    def eighth_start(pos):
        b0 = pos & 1
        b1 = (pos >> 1) & 1
        b2 = (pos >> 2) & 1
        return ((b0 ^ b1) * 4 + b1 * 2 + b2) * eighth

    def body(a_ref, b_ref, out_ref, comm_ref,
             rs_send, rs_recv, ag_send, ag_recv):
        my_pos = lax.axis_index("i")
        my_e = eighth_start(my_pos)

        barrier_sem = pltpu.get_barrier_semaphore()
        for d in PEERS:
            pl.semaphore_signal(
                barrier_sem, inc=1,
                device_id=(my_pos ^ d,), device_id_type=pl.DeviceIdType.MESH,
            )
        pl.semaphore_wait(barrier_sem, len(PEERS))

        pending = []

        rs = {}
        for d in PEERS:
            t = my_pos ^ d
            te = eighth_start(t)
            out_ref[pl.ds(te, eighth), :] = jnp.dot(
                a_ref[pl.ds(te, eighth), :], b_ref[...],
                preferred_element_type=jnp.float32,
            )
            for c in range(N_CHUNK):
                rs[d, c] = pltpu.make_async_remote_copy(
                    src_ref=out_ref.at[pl.ds(te + c * chunk, chunk), :],
                    dst_ref=comm_ref.at[d - 1, pl.ds(c * chunk, chunk), :],
                    send_sem=rs_send.at[d - 1, c],
                    recv_sem=rs_recv.at[d - 1, c],
                    device_id=(t,),
                    device_id_type=pl.DeviceIdType.MESH,
                )
                rs[d, c].start()
                pending.append(rs[d, c])
        out_ref[pl.ds(my_e, eighth), :] = jnp.dot(
            a_ref[pl.ds(my_e, eighth), :], b_ref[...],
            preferred_element_type=jnp.float32,
        )

        for c in range(N_CHUNK):
            row = my_e + c * chunk
            for d in PEERS:
                rs[d, c].wait_recv()
                out_ref[pl.ds(row, chunk), :] += comm_ref[
                    d - 1, pl.ds(c * chunk, chunk), :
                ]
            out_ref[pl.ds(row, chunk), :] = jnp.maximum(
                out_ref[pl.ds(row, chunk), :], 0.0
            )
            for d in PEERS:
                ag = pltpu.make_async_remote_copy(
                    src_ref=out_ref.at[pl.ds(row, chunk), :],
                    dst_ref=out_ref.at[pl.ds(row, chunk), :],
                    send_sem=ag_send.at[d - 1, c],
                    recv_sem=ag_recv.at[d - 1, c],
                    device_id=(my_pos ^ d,),
                    device_id_type=pl.DeviceIdType.MESH,
                )
                ag.start()
                pending.append(ag)

        for d in PEERS:
            te = eighth_start(my_pos ^ d)
            for c in range(N_CHUNK):
                recv = pltpu.make_async_remote_copy(
                    src_ref=out_ref.at[pl.ds(my_e + c * chunk, chunk), :],
                    dst_ref=out_ref.at[pl.ds(te + c * chunk, chunk), :],
                    send_sem=ag_send.at[d - 1, c],
                    recv_sem=ag_recv.at[d - 1, c],
                    device_id=(my_pos ^ d,),
                    device_id_type=pl.DeviceIdType.MESH,
                )
                recv.wait_recv()

        for rdma in pending:
            rdma.wait_send()

    return pl.pallas_call(
        body,
        out_shape=jax.ShapeDtypeStruct((m, n), jnp.float32),
        in_specs=[
            pl.BlockSpec(memory_space=pltpu.VMEM),
            pl.BlockSpec(memory_space=pltpu.VMEM),
        ],
        out_specs=pl.BlockSpec(memory_space=pltpu.VMEM),
        scratch_shapes=[
            pltpu.VMEM((N_DEV - 1, eighth, n), jnp.float32),
            pltpu.SemaphoreType.DMA((N_DEV - 1, N_CHUNK)),
            pltpu.SemaphoreType.DMA((N_DEV - 1, N_CHUNK)),
            pltpu.SemaphoreType.DMA((N_DEV - 1, N_CHUNK)),
            pltpu.SemaphoreType.DMA((N_DEV - 1, N_CHUNK)),
        ],
        compiler_params=pltpu.CompilerParams(collective_id=0),
    )(A, B)


# device time: 21419 ns/iter; 2.1589x vs baseline; 1.0718x over previous
import jax
import jax.numpy as jnp
from jax import lax
from jax.experimental import pallas as pl
from jax.experimental.pallas import tpu as pltpu

N_DEV = 8
PEERS = (6, 2, 5, 7, 1, 3, 4)
N_CHUNK = 2


def kernel(A, B):
    m, k = A.shape
    k2, n = B.shape
    eighth = m // N_DEV
    chunk = eighth // N_CHUNK

    def eighth_start(pos):
        b0 = pos & 1
        b1 = (pos >> 1) & 1
        b2 = (pos >> 2) & 1
        return ((b0 ^ b1) * 4 + b1 * 2 + b2) * eighth

    def body(a_ref, b_ref, out_ref, comm_ref,
             rs_send, rs_recv, ag_send, ag_recv):
        my_pos = lax.axis_index("i")
        my_e = eighth_start(my_pos)

        barrier_sem = pltpu.get_barrier_semaphore()
        for d in PEERS:
            pl.semaphore_signal(
                barrier_sem, inc=1,
                device_id=(my_pos ^ d,), device_id_type=pl.DeviceIdType.MESH,
            )
        pl.semaphore_wait(barrier_sem, len(PEERS))

        pending = []

        rs = {}
        for c in range(N_CHUNK):
            for d in PEERS:
                t = my_pos ^ d
                row = eighth_start(t) + c * chunk
                out_ref[pl.ds(row, chunk), :] = jnp.dot(
                    a_ref[pl.ds(row, chunk), :], b_ref[...],
                    preferred_element_type=jnp.float32,
                )
                rs[d, c] = pltpu.make_async_remote_copy(
                    src_ref=out_ref.at[pl.ds(row, chunk), :],
                    dst_ref=comm_ref.at[d - 1, pl.ds(c * chunk, chunk), :],
                    send_sem=rs_send.at[d - 1, c],
                    recv_sem=rs_recv.at[d - 1, c],
                    device_id=(t,),
                    device_id_type=pl.DeviceIdType.MESH,
                )
                rs[d, c].start()
                pending.append(rs[d, c])
            row = my_e + c * chunk
            out_ref[pl.ds(row, chunk), :] = jnp.dot(
                a_ref[pl.ds(row, chunk), :], b_ref[...],
                preferred_element_type=jnp.float32,
            )

        for c in range(N_CHUNK):
            row = my_e + c * chunk
            for d in PEERS:
                rs[d, c].wait_recv()
                out_ref[pl.ds(row, chunk), :] += comm_ref[
                    d - 1, pl.ds(c * chunk, chunk), :
                ]
            out_ref[pl.ds(row, chunk), :] = jnp.maximum(
                out_ref[pl.ds(row, chunk), :], 0.0
            )
            for d in PEERS:
                ag = pltpu.make_async_remote_copy(
                    src_ref=out_ref.at[pl.ds(row, chunk), :],
                    dst_ref=out_ref.at[pl.ds(row, chunk), :],
                    send_sem=ag_send.at[d - 1, c],
                    recv_sem=ag_recv.at[d - 1, c],
                    device_id=(my_pos ^ d,),
                    device_id_type=pl.DeviceIdType.MESH,
                )
                ag.start()
                pending.append(ag)

        for d in PEERS:
            te = eighth_start(my_pos ^ d)
            for c in range(N_CHUNK):
                recv = pltpu.make_async_remote_copy(
                    src_ref=out_ref.at[pl.ds(my_e + c * chunk, chunk), :],
                    dst_ref=out_ref.at[pl.ds(te + c * chunk, chunk), :],
                    send_sem=ag_send.at[d - 1, c],
                    recv_sem=ag_recv.at[d - 1, c],
                    device_id=(my_pos ^ d,),
                    device_id_type=pl.DeviceIdType.MESH,
                )
                recv.wait_recv()

        for rdma in pending:
            rdma.wait_send()

    return pl.pallas_call(
        body,
        out_shape=jax.ShapeDtypeStruct((m, n), jnp.float32),
        in_specs=[
            pl.BlockSpec(memory_space=pltpu.VMEM),
            pl.BlockSpec(memory_space=pltpu.VMEM),
        ],
        out_specs=pl.BlockSpec(memory_space=pltpu.VMEM),
        scratch_shapes=[
            pltpu.VMEM((N_DEV - 1, eighth, n), jnp.float32),
            pltpu.SemaphoreType.DMA((N_DEV - 1, N_CHUNK)),
            pltpu.SemaphoreType.DMA((N_DEV - 1, N_CHUNK)),
            pltpu.SemaphoreType.DMA((N_DEV - 1, N_CHUNK)),
            pltpu.SemaphoreType.DMA((N_DEV - 1, N_CHUNK)),
        ],
        compiler_params=pltpu.CompilerParams(collective_id=0),
    )(A, B)
